# baseline (speedup 1.0000x reference)
import functools

import jax
import jax.numpy as jnp
from jax import lax
from jax.experimental import pallas as pl
from jax.experimental.pallas import tpu as pltpu
from jax.experimental.pallas import tpu_sc as plsc

_INTERPRET = False


def _cdiv(a, b):
    return (a + b - 1) // b


def _lrelu(x):
    return jnp.where(x >= 0, x, 0.01 * x)


def _mm_nb(x, w, bm=1024):
    M, K = x.shape
    N = w.shape[1]

    def kern(x_ref, w_ref, o_ref):
        o_ref[...] = jnp.dot(x_ref[...], w_ref[...],
                             preferred_element_type=jnp.float32)

    return pl.pallas_call(
        kern,
        grid=(_cdiv(M, bm),),
        in_specs=[pl.BlockSpec((bm, K), lambda i: (i, 0)),
                  pl.BlockSpec((K, N), lambda i: (0, 0))],
        out_specs=pl.BlockSpec((bm, N), lambda i: (i, 0)),
        out_shape=jax.ShapeDtypeStruct((M, N), jnp.float32),
        interpret=_INTERPRET,
    )(x, w)


def _mm2(x1, w1, x2, w2, b, bm=1024):
    M, K1 = x1.shape
    K2 = x2.shape[1]
    N = w1.shape[1]

    def kern(x1_ref, w1_ref, x2_ref, w2_ref, b_ref, o_ref):
        acc = jnp.dot(x1_ref[...], w1_ref[...],
                      preferred_element_type=jnp.float32)
        acc = acc + jnp.dot(x2_ref[...], w2_ref[...],
                            preferred_element_type=jnp.float32)
        acc = acc + b_ref[...]
        o_ref[...] = jnp.where(acc >= 0, acc, 0.01 * acc)

    return pl.pallas_call(
        kern,
        grid=(_cdiv(M, bm),),
        in_specs=[pl.BlockSpec((bm, K1), lambda i: (i, 0)),
                  pl.BlockSpec((K1, N), lambda i: (0, 0)),
                  pl.BlockSpec((bm, K2), lambda i: (i, 0)),
                  pl.BlockSpec((K2, N), lambda i: (0, 0)),
                  pl.BlockSpec((1, N), lambda i: (0, 0))],
        out_specs=pl.BlockSpec((bm, N), lambda i: (i, 0)),
        out_shape=jax.ShapeDtypeStruct((M, N), jnp.float32),
        interpret=_INTERPRET,
    )(x1, w1, x2, w2, b.reshape(1, N))


def _mm2p(x1, w1, part, w2, b, bm=1024):
    M, K1 = x1.shape
    K2 = part.shape[2]
    N = w1.shape[1]

    def kern(x1_ref, w1_ref, p0_ref, p1_ref, w2_ref, b_ref, o_ref):
        acc = jnp.dot(x1_ref[...], w1_ref[...],
                      preferred_element_type=jnp.float32)
        acc = acc + jnp.dot(p0_ref[0] + p1_ref[0], w2_ref[...],
                            preferred_element_type=jnp.float32)
        acc = acc + b_ref[...]
        o_ref[...] = jnp.where(acc >= 0, acc, 0.01 * acc)

    return pl.pallas_call(
        kern,
        grid=(_cdiv(M, bm),),
        in_specs=[pl.BlockSpec((bm, K1), lambda i: (i, 0)),
                  pl.BlockSpec((K1, N), lambda i: (0, 0)),
                  pl.BlockSpec((1, bm, K2), lambda i: (0, i, 0)),
                  pl.BlockSpec((1, bm, K2), lambda i: (1, i, 0)),
                  pl.BlockSpec((K2, N), lambda i: (0, 0)),
                  pl.BlockSpec((1, N), lambda i: (0, 0))],
        out_specs=pl.BlockSpec((bm, N), lambda i: (i, 0)),
        out_shape=jax.ShapeDtypeStruct((M, N), jnp.float32),
        interpret=_INTERPRET,
    )(x1, w1, part, part, w2, b.reshape(1, N))


def _mm_addp(part, w2, add, b, m, bm=1024):
    K2 = part.shape[2]
    N = w2.shape[1]

    def kern(p0_ref, p1_ref, w2_ref, a_ref, b_ref, o_ref):
        acc = jnp.dot(p0_ref[0] + p1_ref[0], w2_ref[...],
                      preferred_element_type=jnp.float32)
        acc = acc + a_ref[...] + b_ref[...]
        o_ref[...] = jnp.where(acc >= 0, acc, 0.01 * acc)

    return pl.pallas_call(
        kern,
        grid=(_cdiv(m, bm),),
        in_specs=[pl.BlockSpec((1, bm, K2), lambda i: (0, i, 0)),
                  pl.BlockSpec((1, bm, K2), lambda i: (1, i, 0)),
                  pl.BlockSpec((K2, N), lambda i: (0, 0)),
                  pl.BlockSpec((bm, N), lambda i: (i, 0)),
                  pl.BlockSpec((1, N), lambda i: (0, 0))],
        out_specs=pl.BlockSpec((bm, N), lambda i: (i, 0)),
        out_shape=jax.ShapeDtypeStruct((m, N), jnp.float32),
        interpret=_INTERPRET,
    )(part, part, w2, add, b.reshape(1, N))


def _asm3(a, b, c, bias, bm=1024):
    M, N = a.shape

    def kern(a_ref, b_ref, c_ref, s_ref, o_ref):
        acc = a_ref[...] + b_ref[...] + c_ref[...] + s_ref[...]
        o_ref[...] = jnp.where(acc >= 0, acc, 0.01 * acc)

    return pl.pallas_call(
        kern,
        grid=(_cdiv(M, bm),),
        in_specs=[pl.BlockSpec((bm, N), lambda i: (i, 0)),
                  pl.BlockSpec((bm, N), lambda i: (i, 0)),
                  pl.BlockSpec((bm, N), lambda i: (i, 0)),
                  pl.BlockSpec((1, N), lambda i: (0, 0))],
        out_specs=pl.BlockSpec((bm, N), lambda i: (i, 0)),
        out_shape=jax.ShapeDtypeStruct((M, N), jnp.float32),
        interpret=_INTERPRET,
    )(a, b, c, bias.reshape(1, N))


def _add2(a, b, bm=1024):
    M, N = a.shape

    def kern(a_ref, b_ref, o_ref):
        acc = a_ref[...] + b_ref[...]
        o_ref[...] = jnp.where(acc >= 0, acc, 0.01 * acc)

    return pl.pallas_call(
        kern,
        grid=(_cdiv(M, bm),),
        in_specs=[pl.BlockSpec((bm, N), lambda i: (i, 0)),
                  pl.BlockSpec((bm, N), lambda i: (i, 0))],
        out_specs=pl.BlockSpec((bm, N), lambda i: (i, 0)),
        out_shape=jax.ShapeDtypeStruct((M, N), jnp.float32),
        interpret=_INTERPRET,
    )(a, b)


def _up_stage(z, w1, b1, w2, b2):
    LATD = z.shape[1]
    H = w1.shape[1]
    N = w2.shape[1]
    w1t = jnp.transpose(w1)
    b1t = b1.reshape(H, 1)
    w2t = jnp.transpose(w2)
    b2t = b2.reshape(N, 1)

    def kern(z_ref, w1_ref, b1_ref, w2_ref, b2_ref, o_ref):
        h = w1_ref[...] * z_ref[...] + b1_ref[...]
        h = jnp.where(h >= 0, h, 0.01 * h)
        o_ref[...] = jnp.dot(w2_ref[...], h,
                             preferred_element_type=jnp.float32) + b2_ref[...]

    return pl.pallas_call(
        kern,
        out_shape=jax.ShapeDtypeStruct((N, LATD), jnp.float32),
        interpret=_INTERPRET,
    )(z, w1t, b1t, w2t, b2t)


def _decoder(x, w1, b1, w2, b2, g, bt, bm=1024):
    M, K = x.shape
    H = w1.shape[1]
    F = w2.shape[1]

    def kern(x_ref, w1_ref, b1_ref, w2_ref, b2_ref, g_ref, bt_ref, o_ref):
        h = jnp.dot(x_ref[...], w1_ref[...],
                    preferred_element_type=jnp.float32) + b1_ref[...]
        h = jnp.where(h >= 0, h, 0.01 * h)
        y = jnp.dot(h, w2_ref[...],
                    preferred_element_type=jnp.float32) + b2_ref[...]
        mu = jnp.mean(y, axis=-1, keepdims=True)
        var = jnp.mean((y - mu) ** 2, axis=-1, keepdims=True)
        o_ref[...] = (y - mu) / jnp.sqrt(var + 1e-5) * g_ref[...] + bt_ref[...]

    return pl.pallas_call(
        kern,
        grid=(_cdiv(M, bm),),
        in_specs=[pl.BlockSpec((bm, K), lambda i: (i, 0)),
                  pl.BlockSpec((K, H), lambda i: (0, 0)),
                  pl.BlockSpec((1, H), lambda i: (0, 0)),
                  pl.BlockSpec((H, F), lambda i: (0, 0)),
                  pl.BlockSpec((1, F), lambda i: (0, 0)),
                  pl.BlockSpec((1, F), lambda i: (0, 0)),
                  pl.BlockSpec((1, F), lambda i: (0, 0))],
        out_specs=pl.BlockSpec((bm, F), lambda i: (i, 0)),
        out_shape=jax.ShapeDtypeStruct((M, F), jnp.float32),
        interpret=_INTERPRET,
    )(x, w1, b1.reshape(1, H), w2, b2.reshape(1, F),
      g.reshape(1, F), bt.reshape(1, F))


_NC, _NS = 2, 16
_NW = _NC * _NS
_CH = 64


def _sc_gather_multi(tables, idxs):
    D = tables[0].shape[1]
    n = len(tables)
    outs = tuple(jax.ShapeDtypeStruct((idx.shape[0], D), jnp.float32)
                 for idx in idxs)
    chunk_counts = [idx.shape[0] // _CH for idx in idxs]
    mesh = plsc.VectorSubcoreMesh(core_axis_name="c", subcore_axis_name="s")

    def body(*refs):
        t_refs = refs[:n]
        i_refs = refs[n:2 * n]
        o_refs = refs[2 * n:3 * n]
        idx_v, rows_v, sem = refs[3 * n:]
        w = lax.axis_index("s") * _NC + lax.axis_index("c")
        for ph in range(n):
            nw = (chunk_counts[ph] - w + _NW - 1) // _NW

            def loop_body(i, _, tbl=t_refs[ph], ir=i_refs[ph], orf=o_refs[ph]):
                base = (w + i * _NW) * _CH
                pltpu.sync_copy(ir.at[pl.ds(base, _CH)], idx_v)
                pltpu.async_copy(tbl.at[idx_v], rows_v, sem).wait()
                pltpu.sync_copy(rows_v, orf.at[pl.ds(base, _CH)])
                return 0

            lax.fori_loop(0, nw, loop_body, 0)

    return pl.kernel(
        body, out_type=outs, mesh=mesh,
        scratch_types=(pltpu.VMEM((_CH,), jnp.int32),
                       pltpu.VMEM((_CH, D), jnp.float32),
                       pltpu.SemaphoreType.DMA),
    )(*tables, *idxs)


def _sc_segsum(vals, seg, n_seg):
    E, D = vals.shape
    W = D if D < 128 else 128
    f = D // W
    R = E * f
    n_rows = _cdiv(n_seg * f, 1024) * 1024
    CHR = 64
    G = R // CHR
    npt = n_rows // _NS
    nz = npt // CHR
    vals_r = vals.reshape(R, W)
    if f > 1:
        seg_x = (seg[:, None] * f
                 + jnp.arange(f, dtype=jnp.int32)[None, :]).reshape(-1)
    else:
        seg_x = seg
    zrow = jnp.zeros((CHR, W), jnp.float32)
    mesh = plsc.VectorSubcoreMesh(core_axis_name="c", subcore_axis_name="s")

    def body(vals_ref, seg_ref, z_ref, out_ref, acc_ref, idx_v, rows_v, sem):
        c = lax.axis_index("c")
        s = lax.axis_index("s")
        w = s * _NC + c

        def zbody(j, _):
            pltpu.sync_copy(z_ref, acc_ref.at[pl.ds(s * npt + j * CHR, CHR)])
            return 0

        lax.fori_loop(0, nz, zbody, 0)
        plsc.subcore_barrier()
        nw = (G - w + _NW - 1) // _NW

        def sbody(i, _):
            base = (w + i * _NW) * CHR
            pltpu.sync_copy(seg_ref.at[pl.ds(base, CHR)], idx_v)
            pltpu.sync_copy(vals_ref.at[pl.ds(base, CHR)], rows_v)
            pltpu.async_copy(rows_v, acc_ref.at[idx_v], sem, add=True).wait()
            return 0

        lax.fori_loop(0, nw, sbody, 0)
        plsc.subcore_barrier()

        def obody(j, _):
            r = s * npt + j * CHR
            pltpu.sync_copy(acc_ref.at[pl.ds(r, CHR)], rows_v)
            pltpu.sync_copy(rows_v, out_ref.at[c, pl.ds(r, CHR)])
            return 0

        lax.fori_loop(0, nz, obody, 0)

    out = pl.kernel(
        body, out_type=jax.ShapeDtypeStruct((2, n_rows, W), jnp.float32),
        mesh=mesh,
        scratch_types=(pltpu.VMEM_SHARED((n_rows, W), jnp.float32),
                       pltpu.VMEM((CHR,), jnp.int32),
                       pltpu.VMEM((CHR, W), jnp.float32),
                       pltpu.SemaphoreType.DMA),
    )(vals_r, seg_x, zrow)
    return out.reshape(2, n_rows // f, D)


def _padrow(a):
    return jnp.concatenate([a, jnp.zeros((8, a.shape[1]), a.dtype)], axis=0)


def _pad64(idx, sentinel):
    m = idx.shape[0]
    mp = _cdiv(m, _CH) * _CH
    if mp == m:
        return idx
    return jnp.concatenate(
        [idx, jnp.full((mp - m,), sentinel, jnp.int32)])


def _mpl_coarse(x, e, src, dst, p, n):
    din = x.shape[1]
    we = p['W_e']
    be = p['b_e']
    wn = p['W_n']
    wn_bot = wn[din:]
    dout = we.shape[1]
    if dout < 128:
        pc = 128 - dout
        we = jnp.pad(we, ((0, 0), (0, pc)))
        be = jnp.pad(be, (0, pc))
        wn_bot = jnp.pad(wn_bot, ((0, pc), (0, 0)))
    sp = _mm_nb(x, we[:din])
    dp = _mm_nb(x, we[din:2 * din])
    ep = _mm_nb(e, we[2 * din:])
    ga, gb = _sc_gather_multi([sp, dp], [src, dst])
    e_new = _asm3(ga, gb, ep, be)
    part = _sc_segsum(e_new, dst, n)
    x_new = _mm2p(x, wn[:din], part, wn_bot, p['b_n'])
    return x_new, e_new


def _mpl_fine(x_c, e_c, src_t, dst_t, e_t, inv_pad, dst_f, p, n_f):
    din = x_c.shape[1]
    we = p['W_e']
    we_e = we[2 * din:]
    if e_c.shape[1] > din:
        we_e = jnp.pad(we_e, ((0, e_c.shape[1] - din), (0, 0)))
    sp = _padrow(_mm_nb(x_c, we[:din]))
    dp = _padrow(_mm_nb(x_c, we[din:2 * din]))
    ep = _padrow(_mm_nb(e_c, we_e))
    wn = p['W_n']
    xt = _padrow(_mm_nb(x_c, wn[:din]))
    ga, gb, gc, xn1 = _sc_gather_multi([sp, dp, ep, xt],
                                       [src_t, dst_t, e_t, inv_pad])
    e_new = _asm3(ga, gb, gc, p['b_e'])
    part = _sc_segsum(e_new, dst_f, n_f)
    x_new = _mm_addp(part, wn[din:], xn1, p['b_n'], n_f)
    return x_new, e_new


def _res_up(x, e, ei_c, ei_f, m_id, e_idx, n_c, n_f, e_f, rp):
    n_cc = x.shape[0]
    e_cc = e.shape[0]
    invm = jnp.full((n_f,), n_cc, jnp.int32).at[m_id].set(
        jnp.arange(n_cc, dtype=jnp.int32))
    invE = jnp.full((e_f,), e_cc, jnp.int32).at[e_idx].set(
        jnp.arange(e_cc, dtype=jnp.int32))
    src_f, dst_f = ei_f[0], ei_f[1]
    src_t = invm[src_f]
    dst_t = invm[dst_f]
    inv_pad = _pad64(invm, n_cc)
    x_skip, _ = _mpl_fine(x, e, src_t, dst_t, invE, inv_pad, dst_f,
                          rp['skip'], n_f)
    x1, e1 = _mpl_coarse(x, e, ei_c[0], ei_c[1], rp['mpl1'], n_c)
    x2, e2 = _mpl_fine(x1, e1, src_t, dst_t, invE, inv_pad, dst_f,
                       rp['mpl2'], n_f)
    return _add2(x2, x_skip), e2


def kernel(z, edge_attr, params, edge_index2, edge_index1, edge_index0,
           m_id1, m_id0, e_idx1, e_idx0):
    p = params
    N2, N1, N0 = 2500, 5000, 10000
    E1, E0 = 80000, 160000
    ei2 = edge_index2.astype(jnp.int32)
    ei1 = edge_index1.astype(jnp.int32)
    ei0 = edge_index0.astype(jnp.int32)

    zr = z.reshape(1, -1)
    x = _up_stage(zr, p['up_W1'], p['up_b1'], p['up_W2'], p['up_b2'])
    e = edge_attr

    x, e = _mpl_coarse(x, e, ei2[0], ei2[1], p['bottom'], N2)
    x, e = _res_up(x, e, ei2, ei1, m_id1, e_idx1, N2, N1, E1, p['r0'])
    x, e = _res_up(x, e, ei1, ei0, m_id0, e_idx0, N1, N0, E0, p['r1'])
    x, e = _mpl_coarse(x, e, ei0[0], ei0[1], p['final'], N0)

    xn = _decoder(x, p['nd_W1'], p['nd_b1'], p['nd_W2'], p['nd_b2'],
                  p['nd_ln_g'], p['nd_ln_b'])
    en = _decoder(e, p['ed_W1'], p['ed_b1'], p['ed_W2'], p['ed_b2'],
                  p['ed_ln_g'], p['ed_ln_b'])
    return xn, en


# speedup vs baseline: 1.0296x; 1.0296x over previous
import functools

import jax
import jax.numpy as jnp
from jax import lax
from jax.experimental import pallas as pl
from jax.experimental.pallas import tpu as pltpu
from jax.experimental.pallas import tpu_sc as plsc

_INTERPRET = False


def _cdiv(a, b):
    return (a + b - 1) // b


def _lrelu(x):
    return jnp.where(x >= 0, x, 0.01 * x)


def _mm_nb(x, w, bm=1024):
    M, K = x.shape
    N = w.shape[1]

    def kern(x_ref, w_ref, o_ref):
        o_ref[...] = jnp.dot(x_ref[...], w_ref[...],
                             preferred_element_type=jnp.float32)

    return pl.pallas_call(
        kern,
        grid=(_cdiv(M, bm),),
        in_specs=[pl.BlockSpec((bm, K), lambda i: (i, 0)),
                  pl.BlockSpec((K, N), lambda i: (0, 0))],
        out_specs=pl.BlockSpec((bm, N), lambda i: (i, 0)),
        out_shape=jax.ShapeDtypeStruct((M, N), jnp.float32),
        interpret=_INTERPRET,
    )(x, w)


def _mm2(x1, w1, x2, w2, b, bm=1024):
    M, K1 = x1.shape
    K2 = x2.shape[1]
    N = w1.shape[1]

    def kern(x1_ref, w1_ref, x2_ref, w2_ref, b_ref, o_ref):
        acc = jnp.dot(x1_ref[...], w1_ref[...],
                      preferred_element_type=jnp.float32)
        acc = acc + jnp.dot(x2_ref[...], w2_ref[...],
                            preferred_element_type=jnp.float32)
        acc = acc + b_ref[...]
        o_ref[...] = jnp.where(acc >= 0, acc, 0.01 * acc)

    return pl.pallas_call(
        kern,
        grid=(_cdiv(M, bm),),
        in_specs=[pl.BlockSpec((bm, K1), lambda i: (i, 0)),
                  pl.BlockSpec((K1, N), lambda i: (0, 0)),
                  pl.BlockSpec((bm, K2), lambda i: (i, 0)),
                  pl.BlockSpec((K2, N), lambda i: (0, 0)),
                  pl.BlockSpec((1, N), lambda i: (0, 0))],
        out_specs=pl.BlockSpec((bm, N), lambda i: (i, 0)),
        out_shape=jax.ShapeDtypeStruct((M, N), jnp.float32),
        interpret=_INTERPRET,
    )(x1, w1, x2, w2, b.reshape(1, N))


def _mm2p(x1, w1, part, w2, b, bm=1024):
    M, K1 = x1.shape
    K2 = part.shape[2]
    N = w1.shape[1]

    def kern(x1_ref, w1_ref, p0_ref, p1_ref, w2_ref, b_ref, o_ref):
        acc = jnp.dot(x1_ref[...], w1_ref[...],
                      preferred_element_type=jnp.float32)
        acc = acc + jnp.dot(p0_ref[0] + p1_ref[0], w2_ref[...],
                            preferred_element_type=jnp.float32)
        acc = acc + b_ref[...]
        o_ref[...] = jnp.where(acc >= 0, acc, 0.01 * acc)

    return pl.pallas_call(
        kern,
        grid=(_cdiv(M, bm),),
        in_specs=[pl.BlockSpec((bm, K1), lambda i: (i, 0)),
                  pl.BlockSpec((K1, N), lambda i: (0, 0)),
                  pl.BlockSpec((1, bm, K2), lambda i: (0, i, 0)),
                  pl.BlockSpec((1, bm, K2), lambda i: (1, i, 0)),
                  pl.BlockSpec((K2, N), lambda i: (0, 0)),
                  pl.BlockSpec((1, N), lambda i: (0, 0))],
        out_specs=pl.BlockSpec((bm, N), lambda i: (i, 0)),
        out_shape=jax.ShapeDtypeStruct((M, N), jnp.float32),
        interpret=_INTERPRET,
    )(x1, w1, part, part, w2, b.reshape(1, N))


def _mm_addp(part, w2, add, b, m, bm=1024):
    K2 = part.shape[2]
    N = w2.shape[1]

    def kern(p0_ref, p1_ref, w2_ref, a_ref, b_ref, o_ref):
        acc = jnp.dot(p0_ref[0] + p1_ref[0], w2_ref[...],
                      preferred_element_type=jnp.float32)
        acc = acc + a_ref[...] + b_ref[...]
        o_ref[...] = jnp.where(acc >= 0, acc, 0.01 * acc)

    return pl.pallas_call(
        kern,
        grid=(_cdiv(m, bm),),
        in_specs=[pl.BlockSpec((1, bm, K2), lambda i: (0, i, 0)),
                  pl.BlockSpec((1, bm, K2), lambda i: (1, i, 0)),
                  pl.BlockSpec((K2, N), lambda i: (0, 0)),
                  pl.BlockSpec((bm, N), lambda i: (i, 0)),
                  pl.BlockSpec((1, N), lambda i: (0, 0))],
        out_specs=pl.BlockSpec((bm, N), lambda i: (i, 0)),
        out_shape=jax.ShapeDtypeStruct((m, N), jnp.float32),
        interpret=_INTERPRET,
    )(part, part, w2, add, b.reshape(1, N))


def _asm3(a, b, c, bias, bm=1024):
    M, N = a.shape

    def kern(a_ref, b_ref, c_ref, s_ref, o_ref):
        acc = a_ref[...] + b_ref[...] + c_ref[...] + s_ref[...]
        o_ref[...] = jnp.where(acc >= 0, acc, 0.01 * acc)

    return pl.pallas_call(
        kern,
        grid=(_cdiv(M, bm),),
        in_specs=[pl.BlockSpec((bm, N), lambda i: (i, 0)),
                  pl.BlockSpec((bm, N), lambda i: (i, 0)),
                  pl.BlockSpec((bm, N), lambda i: (i, 0)),
                  pl.BlockSpec((1, N), lambda i: (0, 0))],
        out_specs=pl.BlockSpec((bm, N), lambda i: (i, 0)),
        out_shape=jax.ShapeDtypeStruct((M, N), jnp.float32),
        interpret=_INTERPRET,
    )(a, b, c, bias.reshape(1, N))


def _add2(a, b, bm=1024):
    M, N = a.shape

    def kern(a_ref, b_ref, o_ref):
        acc = a_ref[...] + b_ref[...]
        o_ref[...] = jnp.where(acc >= 0, acc, 0.01 * acc)

    return pl.pallas_call(
        kern,
        grid=(_cdiv(M, bm),),
        in_specs=[pl.BlockSpec((bm, N), lambda i: (i, 0)),
                  pl.BlockSpec((bm, N), lambda i: (i, 0))],
        out_specs=pl.BlockSpec((bm, N), lambda i: (i, 0)),
        out_shape=jax.ShapeDtypeStruct((M, N), jnp.float32),
        interpret=_INTERPRET,
    )(a, b)


def _up_stage(z, w1, b1, w2, b2):
    LATD = z.shape[1]
    H = w1.shape[1]
    N = w2.shape[1]
    w1t = jnp.transpose(w1)
    b1t = b1.reshape(H, 1)
    w2t = jnp.transpose(w2)
    b2t = b2.reshape(N, 1)

    def kern(z_ref, w1_ref, b1_ref, w2_ref, b2_ref, o_ref):
        h = w1_ref[...] * z_ref[...] + b1_ref[...]
        h = jnp.where(h >= 0, h, 0.01 * h)
        o_ref[...] = jnp.dot(w2_ref[...], h,
                             preferred_element_type=jnp.float32) + b2_ref[...]

    return pl.pallas_call(
        kern,
        out_shape=jax.ShapeDtypeStruct((N, LATD), jnp.float32),
        interpret=_INTERPRET,
    )(z, w1t, b1t, w2t, b2t)


def _decoder(x, w1, b1, w2, b2, g, bt, bm=1024):
    M, K = x.shape
    H = w1.shape[1]
    F = w2.shape[1]

    def kern(x_ref, w1_ref, b1_ref, w2_ref, b2_ref, g_ref, bt_ref, o_ref):
        h = jnp.dot(x_ref[...], w1_ref[...],
                    preferred_element_type=jnp.float32) + b1_ref[...]
        h = jnp.where(h >= 0, h, 0.01 * h)
        y = jnp.dot(h, w2_ref[...],
                    preferred_element_type=jnp.float32) + b2_ref[...]
        mu = jnp.mean(y, axis=-1, keepdims=True)
        var = jnp.mean((y - mu) ** 2, axis=-1, keepdims=True)
        o_ref[...] = (y - mu) / jnp.sqrt(var + 1e-5) * g_ref[...] + bt_ref[...]

    return pl.pallas_call(
        kern,
        grid=(_cdiv(M, bm),),
        in_specs=[pl.BlockSpec((bm, K), lambda i: (i, 0)),
                  pl.BlockSpec((K, H), lambda i: (0, 0)),
                  pl.BlockSpec((1, H), lambda i: (0, 0)),
                  pl.BlockSpec((H, F), lambda i: (0, 0)),
                  pl.BlockSpec((1, F), lambda i: (0, 0)),
                  pl.BlockSpec((1, F), lambda i: (0, 0)),
                  pl.BlockSpec((1, F), lambda i: (0, 0))],
        out_specs=pl.BlockSpec((bm, F), lambda i: (i, 0)),
        out_shape=jax.ShapeDtypeStruct((M, F), jnp.float32),
        interpret=_INTERPRET,
    )(x, w1, b1.reshape(1, H), w2, b2.reshape(1, F),
      g.reshape(1, F), bt.reshape(1, F))


_NC, _NS = 2, 16
_NW = _NC * _NS


def _gchunk(D):
    return 20480 // D


def _sc_gather_multi(tables, idxs):
    D = tables[0].shape[1]
    CH = _gchunk(D)
    n = len(tables)
    outs = tuple(jax.ShapeDtypeStruct((idx.shape[0], D), jnp.float32)
                 for idx in idxs)
    chunk_counts = []
    for idx in idxs:
        assert idx.shape[0] % CH == 0, (idx.shape, CH)
        chunk_counts.append(idx.shape[0] // CH)
    mesh = plsc.VectorSubcoreMesh(core_axis_name="c", subcore_axis_name="s")

    def body(*refs):
        t_refs = refs[:n]
        i_refs = refs[n:2 * n]
        o_refs = refs[2 * n:3 * n]
        idx0, idx1, rows0, rows1, sem = refs[3 * n:]
        w = lax.axis_index("s") * _NC + lax.axis_index("c")
        for ph in range(n):
            nw = (chunk_counts[ph] - w + _NW - 1) // _NW
            tbl, ir, orf = t_refs[ph], i_refs[ph], o_refs[ph]

            def issue(i, slot, tbl=tbl, ir=ir):
                iv = [idx0, idx1][slot]
                rv = [rows0, rows1][slot]
                base = (w + i * _NW) * CH
                pltpu.sync_copy(ir.at[pl.ds(base, CH)], iv)
                pltpu.async_copy(tbl.at[iv], rv, sem.at[slot])

            def drain(i, slot, tbl=tbl, orf=orf):
                iv = [idx0, idx1][slot]
                rv = [rows0, rows1][slot]
                base = (w + i * _NW) * CH
                pltpu.make_async_copy(tbl.at[iv], rv, sem.at[slot]).wait()
                pltpu.sync_copy(rv, orf.at[pl.ds(base, CH)])

            @pl.when(nw > 0)
            def _(issue=issue, drain=drain, nw=nw):
                issue(0, 0)

                def lb(i, _):
                    par = lax.rem(i, 2)

                    @pl.when(par == 0)
                    def _():
                        issue(i, 0)
                        drain(i - 1, 1)

                    @pl.when(par == 1)
                    def _():
                        issue(i, 1)
                        drain(i - 1, 0)

                    return 0

                lax.fori_loop(1, nw, lb, 0)
                last = nw - 1
                lpar = lax.rem(last, 2)

                @pl.when(lpar == 0)
                def _():
                    drain(last, 0)

                @pl.when(lpar == 1)
                def _():
                    drain(last, 1)

    return pl.kernel(
        body, out_type=outs, mesh=mesh,
        scratch_types=(pltpu.VMEM((CH,), jnp.int32),
                       pltpu.VMEM((CH,), jnp.int32),
                       pltpu.VMEM((CH, D), jnp.float32),
                       pltpu.VMEM((CH, D), jnp.float32),
                       pltpu.SemaphoreType.DMA((2,))),
    )(*tables, *idxs)


_CHR = 160
_CHZ = 64


def _sc_segsum(vals, seg, n_seg, tok):
    E, D = vals.shape
    W = 128
    assert D % W == 0, D
    f = D // W
    R = E * f
    n_rows = _cdiv(n_seg * f, 1024) * 1024
    assert R % _CHR == 0, (R, _CHR)
    G = R // _CHR
    npt = n_rows // _NS
    nz = npt // _CHZ
    vals_r = vals.reshape(R, W)
    if f > 1:
        seg_x = (seg[:, None] * f
                 + jnp.arange(f, dtype=jnp.int32)[None, :]).reshape(-1)
    else:
        seg_x = seg
    zrow = jnp.zeros((_CHZ, W), jnp.float32) + 0.0 * tok
    mesh = plsc.VectorSubcoreMesh(core_axis_name="c", subcore_axis_name="s")

    def body(vals_ref, seg_ref, z_ref, out_ref, acc_ref,
             idx0, idx1, rows0, rows1, lsem, ssem, osem):
        c = lax.axis_index("c")
        s = lax.axis_index("s")
        w = s * _NC + c

        def zslice(j):
            return acc_ref.at[pl.ds(s * npt + j * _CHZ, _CHZ)]

        def zbody(j, _):
            pltpu.async_copy(z_ref, zslice(j), osem)
            return 0

        lax.fori_loop(0, nz, zbody, 0)

        def zwait(j, _):
            pltpu.make_async_copy(z_ref, zslice(j), osem).wait()
            return 0

        lax.fori_loop(0, nz, zwait, 0)
        plsc.subcore_barrier()

        nw = (G - w + _NW - 1) // _NW

        def load(i, slot):
            iv = [idx0, idx1][slot]
            rv = [rows0, rows1][slot]
            base = (w + i * _NW) * _CHR
            pltpu.sync_copy(seg_ref.at[pl.ds(base, _CHR)], iv)
            pltpu.async_copy(vals_ref.at[pl.ds(base, _CHR)], rv, lsem.at[slot])

        def scat(i, slot):
            iv = [idx0, idx1][slot]
            rv = [rows0, rows1][slot]
            base = (w + i * _NW) * _CHR
            pltpu.make_async_copy(vals_ref.at[pl.ds(base, _CHR)], rv,
                                  lsem.at[slot]).wait()
            pltpu.async_copy(rv, acc_ref.at[iv], ssem.at[slot], add=True)

        def swait(i, slot):
            iv = [idx0, idx1][slot]
            rv = [rows0, rows1][slot]
            pltpu.make_async_copy(rv, acc_ref.at[iv], ssem.at[slot]).wait()

        @pl.when(nw > 0)
        def _():
            load(0, 0)

            def lb(i, _):
                par = lax.rem(i, 2)

                @pl.when(par == 0)
                def _():
                    scat(i - 1, 1)

                    @pl.when(i >= 2)
                    def _():
                        swait(i - 2, 0)

                    load(i, 0)

                @pl.when(par == 1)
                def _():
                    scat(i - 1, 0)

                    @pl.when(i >= 2)
                    def _():
                        swait(i - 2, 1)

                    load(i, 1)

                return 0

            lax.fori_loop(1, nw, lb, 0)
            last = nw - 1
            lpar = lax.rem(last, 2)

            @pl.when(lpar == 0)
            def _():
                scat(last, 0)

                @pl.when(last >= 1)
                def _():
                    swait(last - 1, 1)

                swait(last, 0)

            @pl.when(lpar == 1)
            def _():
                scat(last, 1)
                swait(last - 1, 0)
                swait(last, 1)

        plsc.subcore_barrier()

        def oslice(j):
            r = s * npt + j * _CHZ
            return (acc_ref.at[pl.ds(r, _CHZ)], out_ref.at[c, pl.ds(r, _CHZ)])

        def obody(j, _):
            src, dst = oslice(j)
            pltpu.async_copy(src, dst, osem)
            return 0

        lax.fori_loop(0, nz, obody, 0)

        def owait(j, _):
            src, dst = oslice(j)
            pltpu.make_async_copy(src, dst, osem).wait()
            return 0

        lax.fori_loop(0, nz, owait, 0)

    out = pl.kernel(
        body, out_type=jax.ShapeDtypeStruct((2, n_rows, W), jnp.float32),
        mesh=mesh,
        scratch_types=(pltpu.VMEM_SHARED((n_rows, W), jnp.float32),
                       pltpu.VMEM((_CHR,), jnp.int32),
                       pltpu.VMEM((_CHR,), jnp.int32),
                       pltpu.VMEM((_CHR, W), jnp.float32),
                       pltpu.VMEM((_CHR, W), jnp.float32),
                       pltpu.SemaphoreType.DMA((2,)),
                       pltpu.SemaphoreType.DMA((2,)),
                       pltpu.SemaphoreType.DMA),
    )(vals_r, seg_x, zrow)
    tok_new = lax.slice(out, (0, 0, 0), (1, 1, 1)).reshape(1, 1)
    return out.reshape(2, n_rows // f, D), tok_new


def _padrow(a):
    return jnp.concatenate([a, jnp.zeros((8, a.shape[1]), a.dtype)], axis=0)


def _padch(idx, sentinel, ch):
    m = idx.shape[0]
    mp = _cdiv(m, ch) * ch
    if mp == m:
        return idx
    return jnp.concatenate(
        [idx, jnp.full((mp - m,), sentinel, jnp.int32)])


def _mpl_coarse(x, e, src, dst, p, n, tok):
    din = x.shape[1]
    we = p['W_e']
    be = p['b_e']
    wn = p['W_n']
    wn_bot = wn[din:]
    dout = we.shape[1]
    if dout < 128:
        pc = 128 - dout
        we = jnp.pad(we, ((0, 0), (0, pc)))
        be = jnp.pad(be, (0, pc))
        wn_bot = jnp.pad(wn_bot, ((0, pc), (0, 0)))
    sp = _mm_nb(x, we[:din])
    dp = _mm_nb(x, we[din:2 * din])
    ep = _mm_nb(e, we[2 * din:])
    ga, gb = _sc_gather_multi([sp, dp], [src, dst])
    e_new = _asm3(ga, gb, ep, be)
    part, tok = _sc_segsum(e_new, dst, n, tok)
    x_new = _mm2p(x, wn[:din], part, wn_bot, p['b_n'])
    return x_new, e_new, tok


def _mpl_fine(x_c, e_c, src_t, dst_t, e_t, inv_pad, dst_f, p, n_f, tok):
    din = x_c.shape[1]
    we = p['W_e']
    we_e = we[2 * din:]
    if e_c.shape[1] > din:
        we_e = jnp.pad(we_e, ((0, e_c.shape[1] - din), (0, 0)))
    sp = _padrow(_mm_nb(x_c, we[:din]))
    dp = _padrow(_mm_nb(x_c, we[din:2 * din]))
    ep = _padrow(_mm_nb(e_c, we_e))
    wn = p['W_n']
    xt = _padrow(_mm_nb(x_c, wn[:din]))
    ga, gb, gc, xn1 = _sc_gather_multi([sp, dp, ep, xt],
                                       [src_t, dst_t, e_t, inv_pad])
    e_new = _asm3(ga, gb, gc, p['b_e'])
    part, tok = _sc_segsum(e_new, dst_f, n_f, tok)
    x_new = _mm_addp(part, wn[din:], xn1, p['b_n'], n_f)
    return x_new, e_new, tok


def _res_up(x, e, ei_c, ei_f, m_id, e_idx, n_c, n_f, e_f, rp, tok):
    n_cc = x.shape[0]
    e_cc = e.shape[0]
    invm = jnp.full((n_f,), n_cc, jnp.int32).at[m_id].set(
        jnp.arange(n_cc, dtype=jnp.int32))
    invE = jnp.full((e_f,), e_cc, jnp.int32).at[e_idx].set(
        jnp.arange(e_cc, dtype=jnp.int32))
    src_f, dst_f = ei_f[0], ei_f[1]
    src_t = invm[src_f]
    dst_t = invm[dst_f]
    inv_pad = _padch(invm, n_cc, _gchunk(rp['skip']['W_e'].shape[1]))
    x_skip, _, tok = _mpl_fine(x, e, src_t, dst_t, invE, inv_pad, dst_f,
                               rp['skip'], n_f, tok)
    x1, e1, tok = _mpl_coarse(x, e, ei_c[0], ei_c[1], rp['mpl1'], n_c, tok)
    x2, e2, tok = _mpl_fine(x1, e1, src_t, dst_t, invE, inv_pad, dst_f,
                            rp['mpl2'], n_f, tok)
    return _add2(x2, x_skip), e2, tok


def kernel(z, edge_attr, params, edge_index2, edge_index1, edge_index0,
           m_id1, m_id0, e_idx1, e_idx0):
    p = params
    N2, N1, N0 = 2500, 5000, 10000
    E1, E0 = 80000, 160000
    ei2 = edge_index2.astype(jnp.int32)
    ei1 = edge_index1.astype(jnp.int32)
    ei0 = edge_index0.astype(jnp.int32)

    zr = z.reshape(1, -1)
    x = _up_stage(zr, p['up_W1'], p['up_b1'], p['up_W2'], p['up_b2'])
    e = edge_attr

    tok = jnp.zeros((1, 1), jnp.float32)
    x, e, tok = _mpl_coarse(x, e, ei2[0], ei2[1], p['bottom'], N2, tok)
    x, e, tok = _res_up(x, e, ei2, ei1, m_id1, e_idx1, N2, N1, E1,
                        p['r0'], tok)
    x, e, tok = _res_up(x, e, ei1, ei0, m_id0, e_idx0, N1, N0, E0,
                        p['r1'], tok)
    x, e, tok = _mpl_coarse(x, e, ei0[0], ei0[1], p['final'], N0, tok)

    xn = _decoder(x, p['nd_W1'], p['nd_b1'], p['nd_W2'], p['nd_b2'],
                  p['nd_ln_g'], p['nd_ln_b'])
    en = _decoder(e, p['ed_W1'], p['ed_b1'], p['ed_W2'], p['ed_b2'],
                  p['ed_ln_g'], p['ed_ln_b'])
    return xn, en
